# Initial kernel scaffold; baseline (speedup 1.0000x reference)
import jax
import jax.numpy as jnp
from jax.experimental import pallas as pl


def kernel(x, annotator_matrix, num_dist_types, num_levels):
    raise NotImplementedError("write your pallas kernel here")


# speedup vs baseline: 2.7025x; 2.7025x over previous
import functools

import jax
import jax.numpy as jnp
from jax.experimental import pallas as pl

_ALPHA = 0.1
_FMIN = float(jnp.finfo(jnp.float32).min)
_FMAX = float(jnp.finfo(jnp.float32).max)


def _body(vm_ref, x_ref, am_ref, o_ref, *, bb, n, total_count):
    b = pl.program_id(0)
    nb = pl.num_programs(0)
    am = am_ref[...]
    vm = vm_ref[...] > 0.5
    vm3 = vm[None]
    tmax = jnp.where(vm3, am, _FMIN)
    tmin = jnp.where(vm3, am, _FMAX)
    mx = jnp.max(tmax, axis=2, keepdims=True)
    mn = jnp.min(tmin, axis=2, keepdims=True)
    jidx = jax.lax.broadcasted_iota(jnp.int32, am.shape, 2)
    jmax = jnp.min(jnp.where((tmax == mx) & vm3, jidx, n), axis=2)
    jmin = jnp.min(jnp.where((tmin == mn) & vm3, jidx, n), axis=2)

    kk = jax.lax.broadcasted_iota(jnp.int32, (n, n), 1)
    total = jnp.zeros((), jnp.float32)
    for s in range(bb):
        xs = x_ref[s]
        xb = xs.astype(jnp.bfloat16)
        g = jax.lax.dot_general(xb, xb, (((1,), (1,)), ((), ())),
                                preferred_element_type=jnp.float32)
        r2 = jnp.sum(xs * xs, axis=1, keepdims=True)
        ohp = kk == jmax[s][:, None]
        ohn = kk == jmin[s][:, None]
        gp = jnp.sum(jnp.where(ohp, g, 0.0), axis=1, keepdims=True)
        gn = jnp.sum(jnp.where(ohn, g, 0.0), axis=1, keepdims=True)
        r2t = r2.reshape(1, n)
        r2p = jnp.sum(jnp.where(ohp, r2t, 0.0), axis=1, keepdims=True)
        r2n = jnp.sum(jnp.where(ohn, r2t, 0.0), axis=1, keepdims=True)
        dp = jnp.sqrt(jnp.maximum(r2 + r2p - 2.0 * gp, 0.0))
        dn = jnp.sqrt(jnp.maximum(r2 + r2n - 2.0 * gn, 0.0))
        total += jnp.sum(jnp.maximum(dp - dn + _ALPHA, 0.0))

    prev = jnp.where(b == 0, 0.0, o_ref[0, 0])
    snew = prev + total
    o_ref[...] = jnp.where(b == nb - 1, snew / total_count, snew).reshape(1, 1)


@jax.jit
def kernel(x, annotator_matrix, num_dist_types, num_levels):
    b, n, d = x.shape
    m = n - 1
    i = jnp.arange(n)[:, None]
    j = jnp.arange(n)[None, :]
    same_block = ((i - 1) // num_levels) == (((j - 1) * num_dist_types) // m)
    valid = jnp.where(i == 0, j > 0, jnp.where(j == 0, True, ~same_block))
    vmask = valid.astype(jnp.float32)

    bb = 8
    grid = b // bb
    out = pl.pallas_call(
        functools.partial(_body, bb=bb, n=n, total_count=b * n),
        grid=(grid,),
        in_specs=[
            pl.BlockSpec((n, n), lambda g: (0, 0)),
            pl.BlockSpec((bb, n, d), lambda g: (g, 0, 0)),
            pl.BlockSpec((bb, n, n), lambda g: (g, 0, 0)),
        ],
        out_specs=pl.BlockSpec((1, 1), lambda g: (0, 0)),
        out_shape=jax.ShapeDtypeStruct((1, 1), jnp.float32),
    )(vmask, x, annotator_matrix)
    return out[0, 0]


# speedup vs baseline: 3.7079x; 1.3720x over previous
import functools

import jax
import jax.numpy as jnp
from jax.experimental import pallas as pl

_ALPHA = 0.1
_FMIN = float(jnp.finfo(jnp.float32).min)
_FMAX = float(jnp.finfo(jnp.float32).max)


def _body(vm_ref, x_ref, am_ref, o_ref, *, bb, n, total_count):
    b = pl.program_id(0)
    nb = pl.num_programs(0)
    am = am_ref[...]
    vm = vm_ref[...] > 0.5
    vm3 = vm[None]
    tmax = jnp.where(vm3, am, _FMIN)
    tmin = jnp.where(vm3, am, _FMAX)
    mx = jnp.max(tmax, axis=2, keepdims=True)
    mn = jnp.min(tmin, axis=2, keepdims=True)
    jidx = jax.lax.broadcasted_iota(jnp.int32, am.shape, 2)
    jmax = jnp.min(jnp.where((tmax == mx) & vm3, jidx, n), axis=2)
    jmin = jnp.min(jnp.where((tmin == mn) & vm3, jidx, n), axis=2)

    xall = x_ref[...]
    xb = xall.astype(jnp.bfloat16)
    g = jax.lax.dot_general(xb, xb, (((2,), (2,)), ((0,), (0,))),
                            preferred_element_type=jnp.float32)
    r2 = jnp.sum(xall * xall, axis=2, keepdims=True)
    kk = jax.lax.broadcasted_iota(jnp.int32, (bb, n, n), 2)
    ohp = kk == jmax[:, :, None]
    ohn = kk == jmin[:, :, None]
    gp = jnp.sum(jnp.where(ohp, g, 0.0), axis=2, keepdims=True)
    gn = jnp.sum(jnp.where(ohn, g, 0.0), axis=2, keepdims=True)
    r2t = r2.reshape(bb, 1, n)
    r2p = jnp.sum(jnp.where(ohp, r2t, 0.0), axis=2, keepdims=True)
    r2n = jnp.sum(jnp.where(ohn, r2t, 0.0), axis=2, keepdims=True)
    dp = jnp.sqrt(jnp.maximum(r2 + r2p - 2.0 * gp, 0.0))
    dn = jnp.sqrt(jnp.maximum(r2 + r2n - 2.0 * gn, 0.0))
    total = jnp.sum(jnp.maximum(dp - dn + _ALPHA, 0.0))

    prev = jnp.where(b == 0, 0.0, o_ref[0, 0])
    snew = prev + total
    o_ref[...] = jnp.where(b == nb - 1, snew / total_count, snew).reshape(1, 1)


@jax.jit
def kernel(x, annotator_matrix, num_dist_types, num_levels):
    b, n, d = x.shape
    m = n - 1
    i = jnp.arange(n)[:, None]
    j = jnp.arange(n)[None, :]
    same_block = ((i - 1) // num_levels) == (((j - 1) * num_dist_types) // m)
    valid = jnp.where(i == 0, j > 0, jnp.where(j == 0, True, ~same_block))
    vmask = valid.astype(jnp.float32)

    bb = 8
    grid = b // bb
    out = pl.pallas_call(
        functools.partial(_body, bb=bb, n=n, total_count=b * n),
        grid=(grid,),
        in_specs=[
            pl.BlockSpec((n, n), lambda g: (0, 0)),
            pl.BlockSpec((bb, n, d), lambda g: (g, 0, 0)),
            pl.BlockSpec((bb, n, n), lambda g: (g, 0, 0)),
        ],
        out_specs=pl.BlockSpec((1, 1), lambda g: (0, 0)),
        out_shape=jax.ShapeDtypeStruct((1, 1), jnp.float32),
    )(vmask, x, annotator_matrix)
    return out[0, 0]
